# baseline (speedup 1.0000x reference)
import functools

import jax
import jax.numpy as jnp
from jax.experimental import pallas as pl
from jax.experimental.pallas import tpu as pltpu
from jax.experimental.pallas import tpu_sc as plsc

_B, _N = 4096, 128
_R = _B * _N
_KIN_DIM = 128
_EMB_DIM = 64
_BB = 32
_BR = _BB * _N

_NW = 32
_RW = _R // _NW
_CH = 128
_NCH = _RW // _CH

_sc_mesh = plsc.VectorSubcoreMesh(core_axis_name="c", subcore_axis_name="s")


@functools.partial(
    pl.kernel,
    out_type=jax.ShapeDtypeStruct((_R, 256), jnp.float32),
    mesh=_sc_mesh,
    scratch_types=[
        pltpu.VMEM((_CH,), jnp.int32),
        pltpu.VMEM((_CH,), jnp.int32),
        pltpu.VMEM((_CH,), jnp.int32),
        pltpu.VMEM((_CH, 128), jnp.float32),
        pltpu.SemaphoreType.DMA,
    ],
)
def _sc_emb(ids_hbm, ch_hbm, tab_hbm, out_hbm, idv, chv, keyv, rows, sem):
    c = jax.lax.axis_index("c")
    s = jax.lax.axis_index("s")
    base = (s * 2 + c) * _RW

    def chunk(i, carry):
        off = base + i * _CH
        pltpu.sync_copy(ids_hbm.at[pl.ds(off, _CH)], idv)
        pltpu.sync_copy(ch_hbm.at[pl.ds(off, _CH)], chv)

        def vec(j, carry2):
            sl = pl.ds(j * 16, 16)
            keyv[sl] = idv[sl] * 3 + chv[sl] + 1
            return carry2

        jax.lax.fori_loop(0, _CH // 16, vec, 0)
        pltpu.async_copy(tab_hbm.at[keyv], rows, sem).wait()
        pltpu.sync_copy(rows, out_hbm.at[pl.ds(off, _CH), pl.ds(128, 128)])
        return carry

    jax.lax.fori_loop(0, _NCH, chunk, 0)


def _tc_body(kin_ref, w_ref, b_ref, emb_ref, out_ref):
    del emb_ref
    for i in range(_BB):
        kin_emb = jax.lax.dot_general(
            kin_ref[i], w_ref[...], (((0,), (0,)), ((), ())),
            preferred_element_type=jnp.float32)
        out_ref[i * _N:(i + 1) * _N, :] = kin_emb + b_ref[...]


@functools.partial(jax.jit, static_argnames=("interpret",))
def _run(kinematics, particle_ids, charges, W, b, pid_table, charge_table,
         interpret=False):
    kin_t = jnp.transpose(kinematics, (0, 2, 1))
    ids = particle_ids.reshape(_R)
    ch = charges.reshape(_R)
    b2 = b.reshape(1, _KIN_DIM)
    k = jnp.arange(60)
    ctab = jnp.zeros((64, 2 * _EMB_DIM), jnp.float32)
    ctab = ctab.at[:60, :_EMB_DIM].set(pid_table[k // 3])
    ctab = ctab.at[:60, _EMB_DIM:].set(charge_table[k % 3])

    emb_out = _sc_emb(ids, ch, ctab)

    out = pl.pallas_call(
        _tc_body,
        grid=(_B // _BB,),
        in_specs=[
            pl.BlockSpec((_BB, 4, _N), lambda i: (i, 0, 0)),
            pl.BlockSpec((4, _KIN_DIM), lambda i: (0, 0)),
            pl.BlockSpec((1, _KIN_DIM), lambda i: (0, 0)),
            pl.BlockSpec(memory_space=pltpu.MemorySpace.HBM),
        ],
        out_specs=pl.BlockSpec((_BR, _KIN_DIM), lambda i: (i, 0)),
        out_shape=jax.ShapeDtypeStruct((_R, 256), jnp.float32),
        input_output_aliases={3: 0},
        compiler_params=pltpu.CompilerParams(
            dimension_semantics=("parallel",)),
        interpret=interpret,
    )(kin_t, W, b2, emb_out)
    return out.reshape(_B, _N, 256)


def kernel(kinematics, particle_ids, charges, W, b, pid_table, charge_table):
    return _run(kinematics, particle_ids, charges, W, b, pid_table,
                charge_table)


# speedup vs baseline: 1.0051x; 1.0051x over previous
import functools

import jax
import jax.numpy as jnp
from jax.experimental import pallas as pl
from jax.experimental.pallas import tpu as pltpu
from jax.experimental.pallas import tpu_sc as plsc

_B, _N = 4096, 128
_R = _B * _N
_KIN_DIM = 128
_EMB_DIM = 64
_BB = 32
_BR = _BB * _N

_NW = 32
_RW = _R // _NW
_CH = 128
_NCH = _RW // _CH
_NBUF = 5
_LOOK = 2

_sc_mesh = plsc.VectorSubcoreMesh(core_axis_name="c", subcore_axis_name="s")


@functools.partial(
    pl.kernel,
    out_type=jax.ShapeDtypeStruct((_R, 256), jnp.float32),
    mesh=_sc_mesh,
    scratch_types=[
        pltpu.VMEM((_RW,), jnp.int32),
        pltpu.VMEM((_RW,), jnp.int32),
        pltpu.VMEM((_NBUF, _CH, 128), jnp.float32),
        pltpu.SemaphoreType.DMA,
        pltpu.SemaphoreType.DMA,
    ],
)
def _sc_emb(ids_hbm, ch_hbm, tab_hbm, out_hbm, idv, chv, rows, sem_g, sem_o):
    c = jax.lax.axis_index("c")
    s = jax.lax.axis_index("s")
    base = (s * 2 + c) * _RW

    pltpu.sync_copy(ids_hbm.at[pl.ds(base, _RW)], idv)
    pltpu.sync_copy(ch_hbm.at[pl.ds(base, _RW)], chv)

    def keys(i, carry):
        for k in range(8):
            sl = pl.ds(i * 128 + k * 16, 16)
            idv[sl] = idv[sl] * 3 + chv[sl] + 1
        return carry

    jax.lax.fori_loop(0, _RW // 128, keys, 0)

    def gather(j, buf):
        idx = idv.at[pl.ds(j * _CH, _CH)]
        return pltpu.make_async_copy(tab_hbm.at[idx], rows.at[buf], sem_g)

    def outcp(j, buf):
        dst = out_hbm.at[pl.ds(base + j * _CH, _CH), pl.ds(128, 128)]
        return pltpu.make_async_copy(rows.at[buf], dst, sem_o)

    for j in range(_LOOK):
        gather(j, j % _NBUF).start()

    def step(j, carry):
        b = jax.lax.rem(j, _NBUF)
        bn = jax.lax.rem(j + _LOOK, _NBUF)
        gather(j, b).wait()
        outcp(j, b).start()

        @pl.when(j >= _NBUF - _LOOK)
        def _():
            outcp(j - (_NBUF - _LOOK), bn).wait()

        @pl.when(j + _LOOK < _NCH)
        def _():
            gather(j + _LOOK, bn).start()

        return carry

    jax.lax.fori_loop(0, _NCH, step, 0)
    for t in range(_NBUF - _LOOK):
        j = _NCH - (_NBUF - _LOOK) + t
        outcp(j, (j + _LOOK) % _NBUF).wait()


def _tc_body(kin_ref, w_ref, b_ref, emb_ref, out_ref):
    del emb_ref
    for i in range(_BB):
        kin_emb = jax.lax.dot_general(
            kin_ref[i], w_ref[...], (((0,), (0,)), ((), ())),
            preferred_element_type=jnp.float32)
        out_ref[i * _N:(i + 1) * _N, :] = kin_emb + b_ref[...]


@functools.partial(jax.jit, static_argnames=("interpret",))
def _run(kinematics, particle_ids, charges, W, b, pid_table, charge_table,
         interpret=False):
    kin_t = jnp.transpose(kinematics, (0, 2, 1))
    ids = particle_ids.reshape(_R)
    ch = charges.reshape(_R)
    b2 = b.reshape(1, _KIN_DIM)
    k = jnp.arange(60)
    ctab = jnp.zeros((64, 2 * _EMB_DIM), jnp.float32)
    ctab = ctab.at[:60, :_EMB_DIM].set(pid_table[k // 3])
    ctab = ctab.at[:60, _EMB_DIM:].set(charge_table[k % 3])

    emb_out = _sc_emb(ids, ch, ctab)

    out = pl.pallas_call(
        _tc_body,
        grid=(_B // _BB,),
        in_specs=[
            pl.BlockSpec((_BB, 4, _N), lambda i: (i, 0, 0)),
            pl.BlockSpec((4, _KIN_DIM), lambda i: (0, 0)),
            pl.BlockSpec((1, _KIN_DIM), lambda i: (0, 0)),
            pl.BlockSpec(memory_space=pltpu.MemorySpace.HBM),
        ],
        out_specs=pl.BlockSpec((_BR, _KIN_DIM), lambda i: (i, 0)),
        out_shape=jax.ShapeDtypeStruct((_R, 256), jnp.float32),
        input_output_aliases={3: 0},
        compiler_params=pltpu.CompilerParams(
            dimension_semantics=("parallel",)),
        interpret=interpret,
    )(kin_t, W, b2, emb_out)
    return out.reshape(_B, _N, 256)


def kernel(kinematics, particle_ids, charges, W, b, pid_table, charge_table):
    return _run(kinematics, particle_ids, charges, W, b, pid_table,
                charge_table)
